# baseline (speedup 1.0000x reference)
import functools

import jax
import jax.numpy as jnp
from jax import lax
from jax.experimental import pallas as pl
from jax.experimental.pallas import tpu as pltpu
from jax.experimental.pallas import tpu_sc as plsc

N_NODES = 10000
NP = 10240
E = 160000
EP = 163840
D_IN = 256
D_HID = 512
N_CLS = 128
CH = 128
EB_FULL = 32
EB_SPLIT = 32
DB = 128
NT = 16
RPT = NP // NT
NBUF = 4
BM = 512

_MESH = plsc.VectorSubcoreMesh(core_axis_name="c", subcore_axis_name="s")


def _fill1d(ref, n, value):
    v = jnp.full((16,), value, jnp.float32)

    def body(i, carry):
        ref[pl.ds(i * 16, 16)] = v
        return carry

    lax.fori_loop(0, n // 16, body, 0)


def _fill2d_zero(ref, rows, cols):
    z = jnp.zeros((16,), jnp.float32)
    per_row = cols // 16

    def body(i, carry):
        r = i // per_row
        k = i % per_row
        ref[r, pl.ds(k * 16, 16)] = z
        return carry

    lax.fori_loop(0, rows * per_row, body, 0)


def _deg_body(src_hbm, dst_hbm, out_s, out_d, idxbuf, valbuf, obuf, acc):
    c = lax.axis_index("c")
    s = lax.axis_index("s")
    _fill1d(valbuf, DB, 1.0)
    _fill1d(obuf, RPT, 0.0)
    pltpu.sync_copy(obuf, acc.at[pl.ds(s * RPT, RPT)])

    @pl.when(c == 0)
    def _():
        pltpu.sync_copy(src_hbm.at[s], idxbuf)

    @pl.when(c == 1)
    def _():
        pltpu.sync_copy(dst_hbm.at[s], idxbuf)

    plsc.subcore_barrier()

    def batch(b, carry):
        pltpu.sync_copy(valbuf, acc.at[idxbuf.at[b]], add=True)
        return carry

    lax.fori_loop(0, EP // NT // DB, batch, 0)
    plsc.subcore_barrier()
    pltpu.sync_copy(acc.at[pl.ds(s * RPT, RPT)], obuf)

    @pl.when(c == 0)
    def _():
        pltpu.sync_copy(obuf, out_s.at[pl.ds(s * RPT, RPT)])

    @pl.when(c == 1)
    def _():
        pltpu.sync_copy(obuf, out_d.at[pl.ds(s * RPT, RPT)])


_deg_call = pl.kernel(
    _deg_body,
    out_type=[jax.ShapeDtypeStruct((NP,), jnp.float32)] * 2,
    mesh=_MESH,
    scratch_types=[
        pltpu.VMEM((EP // NT // DB, DB), jnp.int32),
        pltpu.VMEM((DB,), jnp.float32),
        pltpu.VMEM((RPT,), jnp.float32),
        pltpu.VMEM_SHARED((NP,), jnp.float32),
    ],
)


def _make_agg(ncpc, split):
    nh = 1 if split else 2 * ncpc
    nout = 2 if split else 2 * ncpc
    eb = EB_SPLIT if split else EB_FULL
    nb = (EP // 32 if split else EP // NT) // eb
    sec_n = 16 if split else 32
    nsec = nb // sec_n
    nbuf = 6

    def body(*refs):
        src3, dst3 = refs[0], refs[1]
        hs = refs[2:2 + nh]
        outs = refs[2 + nh:2 + nh + nout]
        rest = list(refs[2 + nh + nout:])
        idxsets = ((rest[0], rest[1]), (rest[2], rest[3]))
        i = 4
        rows = tuple(rest[i:i + nbuf])
        i += nbuf
        isem = rest[i]
        i += 1
        zsem = rest[i]
        i += 1
        gsem = tuple(rest[i:i + nbuf])
        i += nbuf
        ssem = tuple(rest[i:i + nbuf])
        i += nbuf
        osem = tuple(rest[i:i + 2])
        acc = rest[i + 2]
        c = lax.axis_index("c")
        s = lax.axis_index("s")

        if split:
            eslice = c * NT + s
        else:
            eslice = s

        def idx_load(sec, bufset, sync):
            isx, idx = idxsets[bufset]
            if sync:
                pltpu.sync_copy(src3.at[eslice, pl.ds(sec * sec_n, sec_n)], isx)
                pltpu.sync_copy(dst3.at[eslice, pl.ds(sec * sec_n, sec_n)], idx)
            else:
                pltpu.async_copy(src3.at[eslice, pl.ds(sec * sec_n, sec_n)], isx, isem)
                pltpu.async_copy(dst3.at[eslice, pl.ds(sec * sec_n, sec_n)], idx, isem)

        def idx_drain(bufset):
            isx, idx = idxsets[bufset]
            pltpu.make_async_copy(src3.at[eslice, pl.ds(0, sec_n)], isx, isem).wait()
            pltpu.make_async_copy(dst3.at[eslice, pl.ds(0, sec_n)], idx, isem).wait()

        def zero_acc():
            _fill2d_zero(rows[0], eb, CH)
            for j in range(RPT // eb):
                pltpu.async_copy(rows[0], acc.at[pl.ds(s * RPT + j * eb, eb)], zsem)
            for j in range(RPT // eb):
                pltpu.make_async_copy(rows[0], acc.at[pl.ds(s * RPT + j * eb, eb)], zsem).wait()

        def run_chunk(h):
            def g_start(isx, b, k):
                pltpu.async_copy(h.at[isx.at[b]], rows[k], gsem[k])

            def g_wait(isx, b, k):
                pltpu.make_async_copy(h.at[isx.at[b]], rows[k], gsem[k]).wait()

            def s_start(idx, b, k):
                pltpu.async_copy(rows[k], acc.at[idx.at[b]], ssem[k], add=True)

            def s_wait(idx, b, k):
                pltpu.make_async_copy(rows[k], acc.at[idx.at[b]], ssem[k]).wait()

            ds = nbuf // 2

            def process(bufset, n_batches):
                isx, idx = idxsets[bufset]
                for k in range(nbuf):
                    g_start(isx, k, k)
                for b in range(n_batches):
                    k = b % nbuf
                    g_wait(isx, b, k)
                    s_start(idx, b, k)
                    x = b - ds
                    if x >= 0 and x + nbuf < n_batches:
                        kx = x % nbuf
                        s_wait(idx, x, kx)
                        g_start(isx, x + nbuf, kx)
                for x in range(max(0, n_batches - nbuf), n_batches):
                    s_wait(idx, x, x % nbuf)

            idx_load(0, 0, True)

            def pair(p, carry):
                sec_b = 2 * p + 1
                idx_load(sec_b, 1, False)
                process(0, sec_n)
                idx_drain(1)

                @pl.when(p + 1 < nsec // 2)
                def _():
                    idx_load(sec_b + 1, 0, False)

                process(1, sec_n)

                @pl.when(p + 1 < nsec // 2)
                def _():
                    idx_drain(0)

                return carry

            lax.fori_loop(0, nsec // 2, pair, 0)

        def write_out(out):
            def o_slot(j):
                r = s * RPT + j * eb
                return pl.ds(r, eb)

            for j in range(RPT // eb):
                k = j % 2
                if j >= 2:
                    pltpu.make_async_copy(rows[k], out.at[o_slot(j - 2)], osem[k]).wait()
                pltpu.sync_copy(acc.at[o_slot(j)], rows[k])
                pltpu.async_copy(rows[k], out.at[o_slot(j)], osem[k])
            for j in range(RPT // eb - 2, RPT // eb):
                k = j % 2
                pltpu.make_async_copy(rows[k], out.at[o_slot(j)], osem[k]).wait()

        for j in range(ncpc):
            zero_acc()
            plsc.subcore_barrier()
            if split:
                run_chunk(hs[0])
            else:
                @pl.when(c == 0)
                def _():
                    run_chunk(hs[j])

                @pl.when(c == 1)
                def _():
                    run_chunk(hs[ncpc + j])
            plsc.subcore_barrier()

            if split:
                @pl.when(c == 0)
                def _():
                    write_out(outs[0])

                @pl.when(c == 1)
                def _():
                    write_out(outs[1])
            else:
                @pl.when(c == 0)
                def _():
                    write_out(outs[j])

                @pl.when(c == 1)
                def _():
                    write_out(outs[ncpc + j])

    return pl.kernel(
        body,
        out_type=[jax.ShapeDtypeStruct((NP, CH), jnp.float32)] * nout,
        mesh=_MESH,
        scratch_types=(
            [pltpu.VMEM((sec_n, eb), jnp.int32)] * 4
            + [pltpu.VMEM((eb, CH), jnp.float32)] * nbuf
            + [pltpu.SemaphoreType.DMA] * (4 + 2 * nbuf)
            + [pltpu.VMEM_SHARED((NP, CH), jnp.float32)]
        ),
    )


_agg_l1 = _make_agg(1, False)
_agg_l2 = _make_agg(2, False)
_agg_l3 = _make_agg(1, True)


def _rsq(d):
    return lax.rsqrt(jnp.maximum(d, 1.0))


def _ln_relu_rows(t, g, b):
    mu = jnp.mean(t, axis=1, keepdims=True)
    var = jnp.mean((t - mu) ** 2, axis=1, keepdims=True)
    y = (t - mu) * lax.rsqrt(var + 1e-5) * g + b
    return jnp.maximum(y, 0.0)


def _prep_body(x_ref, ds_ref, o0, o1):
    ro = _rsq(ds_ref[...])
    h = x_ref[...] * ro
    o0[...] = h[:, :CH]
    o1[...] = h[:, CH:]


def _l1_body(a0, a1, w, b, g, be, ds, dd, o0, o1, o2, o3):
    agg = jnp.concatenate([a0[...], a1[...]], axis=1)
    rin = _rsq(dd[...])
    ro = _rsq(ds[...])
    t = jnp.dot(agg, w[...], preferred_element_type=jnp.float32) * rin + b[...]
    h = _ln_relu_rows(t, g[...], be[...]) * ro
    o0[...] = h[:, 0 * CH:1 * CH]
    o1[...] = h[:, 1 * CH:2 * CH]
    o2[...] = h[:, 2 * CH:3 * CH]
    o3[...] = h[:, 3 * CH:4 * CH]


def _l2_body(a0, a1, a2, a3, w2, b2, g2, be2, w3, ds, dd, op):
    agg = jnp.concatenate([a0[...], a1[...], a2[...], a3[...]], axis=1)
    rin = _rsq(dd[...])
    ro = _rsq(ds[...])
    t = jnp.dot(agg, w2[...], preferred_element_type=jnp.float32) * rin + b2[...]
    h = _ln_relu_rows(t, g2[...], be2[...]) * ro
    op[...] = jnp.dot(h, w3[...], preferred_element_type=jnp.float32)


def _final_body(p0, p1, b3, dd, o):
    rin = _rsq(dd[...])
    t = (p0[...] + p1[...]) * rin + b3[...]
    m = jnp.max(t, axis=1, keepdims=True)
    lse = jnp.log(jnp.sum(jnp.exp(t - m), axis=1, keepdims=True)) + m
    o[...] = t - lse


def _row_spec(d):
    return pl.BlockSpec((BM, d), lambda i: (i, 0))


def _full_spec(shape):
    return pl.BlockSpec(shape, lambda i: tuple(0 for _ in shape))


_COL = pl.BlockSpec((BM, 1), lambda i: (i, 0))
_GRID = (NP // BM,)


def _sds(shape):
    return jax.ShapeDtypeStruct(shape, jnp.float32)


_prep_call = pl.pallas_call(
    _prep_body,
    grid=_GRID,
    in_specs=[_row_spec(D_IN), _COL],
    out_specs=[_row_spec(CH), _row_spec(CH)],
    out_shape=[_sds((NP, CH))] * 2,
)

_l1_call = pl.pallas_call(
    _l1_body,
    grid=_GRID,
    in_specs=[_row_spec(CH), _row_spec(CH), _full_spec((D_IN, D_HID)),
              _full_spec((1, D_HID)), _full_spec((1, D_HID)),
              _full_spec((1, D_HID)), _COL, _COL],
    out_specs=[_row_spec(CH)] * 4,
    out_shape=[_sds((NP, CH))] * 4,
)

_l2_call = pl.pallas_call(
    _l2_body,
    grid=_GRID,
    in_specs=[_row_spec(CH)] * 4 + [_full_spec((D_HID, D_HID)),
              _full_spec((1, D_HID)), _full_spec((1, D_HID)),
              _full_spec((1, D_HID)), _full_spec((D_HID, N_CLS)),
              _COL, _COL],
    out_specs=_row_spec(N_CLS),
    out_shape=_sds((NP, N_CLS)),
)

_final_call = pl.pallas_call(
    _final_body,
    grid=_GRID,
    in_specs=[_row_spec(N_CLS), _row_spec(N_CLS),
              _full_spec((1, N_CLS)), _COL],
    out_specs=_row_spec(N_CLS),
    out_shape=_sds((NP, N_CLS)),
)


def kernel(x, edge_index, W1, b1, g1, be1, W2, b2, g2, be2, W3, b3):
    src = edge_index[0].astype(jnp.int32)
    dst = edge_index[1].astype(jnp.int32)
    pad = jnp.full((EP - E,), NP - 1, jnp.int32)
    srcp = jnp.concatenate([src, pad])
    dstp = jnp.concatenate([dst, pad])
    src16d = srcp.reshape(NT, EP // NT // DB, DB)
    dst16d = dstp.reshape(NT, EP // NT // DB, DB)
    src16 = srcp.reshape(NT, EP // NT // EB_FULL, EB_FULL)
    dst16 = dstp.reshape(NT, EP // NT // EB_FULL, EB_FULL)
    src32 = srcp.reshape(32, EP // 32 // EB_SPLIT, EB_SPLIT)
    dst32 = dstp.reshape(32, EP // 32 // EB_SPLIT, EB_SPLIT)
    xp = jnp.pad(x, ((0, NP - N_NODES), (0, 0)))

    deg_s, deg_d = _deg_call(src16d, dst16d)
    ds2 = deg_s.reshape(NP, 1)
    dd2 = deg_d.reshape(NP, 1)

    h0c0, h0c1 = _prep_call(xp, ds2)
    a10, a11 = _agg_l1(src16, dst16, h0c0, h0c1)
    hs = _l1_call(a10, a11, W1, b1.reshape(1, -1), g1.reshape(1, -1),
                  be1.reshape(1, -1), ds2, dd2)
    a2 = _agg_l2(src16, dst16, *hs)
    p = _l2_call(*a2, W2, b2.reshape(1, -1), g2.reshape(1, -1),
                 be2.reshape(1, -1), W3, ds2, dd2)
    p0, p1 = _agg_l3(src32, dst32, p)
    outp = _final_call(p0, p1, b3.reshape(1, -1), dd2)
    return outp[:N_NODES]


# speedup vs baseline: 1.0280x; 1.0280x over previous
import functools

import jax
import jax.numpy as jnp
from jax import lax
from jax.experimental import pallas as pl
from jax.experimental.pallas import tpu as pltpu
from jax.experimental.pallas import tpu_sc as plsc

N_NODES = 10000
NP = 10240
E = 160000
EP = 163840
D_IN = 256
D_HID = 512
N_CLS = 128
CH = 128
EB_FULL = 32
EB_SPLIT = 32
DB = 128
NT = 16
RPT = NP // NT
NBUF = 4
BM = 512

_MESH = plsc.VectorSubcoreMesh(core_axis_name="c", subcore_axis_name="s")


def _fill1d(ref, n, value):
    v = jnp.full((16,), value, jnp.float32)

    def body(i, carry):
        ref[pl.ds(i * 16, 16)] = v
        return carry

    lax.fori_loop(0, n // 16, body, 0)


def _fill2d_zero(ref, rows, cols):
    z = jnp.zeros((16,), jnp.float32)
    per_row = cols // 16

    def body(i, carry):
        r = i // per_row
        k = i % per_row
        ref[r, pl.ds(k * 16, 16)] = z
        return carry

    lax.fori_loop(0, rows * per_row, body, 0)


def _deg_body(src_hbm, dst_hbm, out_s, out_d, idxbuf, valbuf, obuf, acc):
    c = lax.axis_index("c")
    s = lax.axis_index("s")
    _fill1d(valbuf, DB, 1.0)
    _fill1d(obuf, RPT, 0.0)
    pltpu.sync_copy(obuf, acc.at[pl.ds(s * RPT, RPT)])

    @pl.when(c == 0)
    def _():
        pltpu.sync_copy(src_hbm.at[s], idxbuf)

    @pl.when(c == 1)
    def _():
        pltpu.sync_copy(dst_hbm.at[s], idxbuf)

    plsc.subcore_barrier()

    def batch(b, carry):
        pltpu.sync_copy(valbuf, acc.at[idxbuf.at[b]], add=True)
        return carry

    lax.fori_loop(0, EP // NT // DB, batch, 0)
    plsc.subcore_barrier()
    pltpu.sync_copy(acc.at[pl.ds(s * RPT, RPT)], obuf)

    @pl.when(c == 0)
    def _():
        pltpu.sync_copy(obuf, out_s.at[pl.ds(s * RPT, RPT)])

    @pl.when(c == 1)
    def _():
        pltpu.sync_copy(obuf, out_d.at[pl.ds(s * RPT, RPT)])


_deg_call = pl.kernel(
    _deg_body,
    out_type=[jax.ShapeDtypeStruct((NP,), jnp.float32)] * 2,
    mesh=_MESH,
    scratch_types=[
        pltpu.VMEM((EP // NT // DB, DB), jnp.int32),
        pltpu.VMEM((DB,), jnp.float32),
        pltpu.VMEM((RPT,), jnp.float32),
        pltpu.VMEM_SHARED((NP,), jnp.float32),
    ],
)


def _make_agg(ncpc, split):
    nh = 1 if split else 2 * ncpc
    nout = 2 if split else 2 * ncpc
    eb = EB_SPLIT if split else EB_FULL
    nb = (EP // 32 if split else EP // NT) // eb
    sec_n = 16 if split else 32
    nsec = nb // sec_n
    nbuf = 6

    def body(*refs):
        src3, dst3 = refs[0], refs[1]
        hs = refs[2:2 + nh]
        outs = refs[2 + nh:2 + nh + nout]
        rest = list(refs[2 + nh + nout:])
        idxsets = ((rest[0], rest[1]), (rest[2], rest[3]))
        i = 4
        rows = tuple(rest[i:i + nbuf])
        i += nbuf
        isem = rest[i]
        i += 1
        zsem = rest[i]
        i += 1
        gsem = tuple(rest[i:i + nbuf])
        i += nbuf
        ssem = tuple(rest[i:i + nbuf])
        i += nbuf
        osem = tuple(rest[i:i + 2])
        acc = rest[i + 2]
        c = lax.axis_index("c")
        s = lax.axis_index("s")

        if split:
            eslice = c * NT + s
        else:
            eslice = s

        def idx_load(sec, bufset, sync):
            isx, idx = idxsets[bufset]
            if sync:
                pltpu.sync_copy(src3.at[eslice, pl.ds(sec * sec_n, sec_n)], isx)
                pltpu.sync_copy(dst3.at[eslice, pl.ds(sec * sec_n, sec_n)], idx)
            else:
                pltpu.async_copy(src3.at[eslice, pl.ds(sec * sec_n, sec_n)], isx, isem)
                pltpu.async_copy(dst3.at[eslice, pl.ds(sec * sec_n, sec_n)], idx, isem)

        def idx_drain(bufset):
            isx, idx = idxsets[bufset]
            pltpu.make_async_copy(src3.at[eslice, pl.ds(0, sec_n)], isx, isem).wait()
            pltpu.make_async_copy(dst3.at[eslice, pl.ds(0, sec_n)], idx, isem).wait()

        def zero_acc():
            _fill2d_zero(rows[0], eb, CH)
            for j in range(RPT // eb):
                pltpu.async_copy(rows[0], acc.at[pl.ds(s * RPT + j * eb, eb)], zsem)
            for j in range(RPT // eb):
                pltpu.make_async_copy(rows[0], acc.at[pl.ds(s * RPT + j * eb, eb)], zsem).wait()

        def run_chunk(h):
            def g_start(isx, b, k):
                pltpu.async_copy(h.at[isx.at[b]], rows[k], gsem[k])

            def g_wait(isx, b, k):
                pltpu.make_async_copy(h.at[isx.at[b]], rows[k], gsem[k]).wait()

            def s_start(idx, b, k):
                pltpu.async_copy(rows[k], acc.at[idx.at[b]], ssem[k], add=True)

            def s_wait(idx, b, k):
                pltpu.make_async_copy(rows[k], acc.at[idx.at[b]], ssem[k]).wait()

            def process(bufset, n_batches):
                isx, idx = idxsets[bufset]
                for k in range(nbuf):
                    g_start(isx, k, k)
                for b in range(n_batches):
                    k = b % nbuf
                    g_wait(isx, b, k)
                    s_start(idx, b, k)
                    bp = b - 1
                    if bp >= 0 and bp + nbuf < n_batches:
                        kp = bp % nbuf
                        s_wait(idx, bp, kp)
                        g_start(isx, bp + nbuf, kp)
                for bp in range(max(0, n_batches - nbuf), n_batches):
                    s_wait(idx, bp, bp % nbuf)

            idx_load(0, 0, True)

            def pair(p, carry):
                sec_b = 2 * p + 1
                idx_load(sec_b, 1, False)
                process(0, sec_n)
                idx_drain(1)

                @pl.when(p + 1 < nsec // 2)
                def _():
                    idx_load(sec_b + 1, 0, False)

                process(1, sec_n)

                @pl.when(p + 1 < nsec // 2)
                def _():
                    idx_drain(0)

                return carry

            lax.fori_loop(0, nsec // 2, pair, 0)

        def write_out(out):
            def o_slot(j):
                r = s * RPT + j * eb
                return pl.ds(r, eb)

            for j in range(RPT // eb):
                k = j % 2
                if j >= 2:
                    pltpu.make_async_copy(rows[k], out.at[o_slot(j - 2)], osem[k]).wait()
                pltpu.sync_copy(acc.at[o_slot(j)], rows[k])
                pltpu.async_copy(rows[k], out.at[o_slot(j)], osem[k])
            for j in range(RPT // eb - 2, RPT // eb):
                k = j % 2
                pltpu.make_async_copy(rows[k], out.at[o_slot(j)], osem[k]).wait()

        for j in range(ncpc):
            zero_acc()
            plsc.subcore_barrier()
            if split:
                run_chunk(hs[0])
            else:
                @pl.when(c == 0)
                def _():
                    run_chunk(hs[j])

                @pl.when(c == 1)
                def _():
                    run_chunk(hs[ncpc + j])
            plsc.subcore_barrier()

            if split:
                @pl.when(c == 0)
                def _():
                    write_out(outs[0])

                @pl.when(c == 1)
                def _():
                    write_out(outs[1])
            else:
                @pl.when(c == 0)
                def _():
                    write_out(outs[j])

                @pl.when(c == 1)
                def _():
                    write_out(outs[ncpc + j])

    return pl.kernel(
        body,
        out_type=[jax.ShapeDtypeStruct((NP, CH), jnp.float32)] * nout,
        mesh=_MESH,
        scratch_types=(
            [pltpu.VMEM((sec_n, eb), jnp.int32)] * 4
            + [pltpu.VMEM((eb, CH), jnp.float32)] * nbuf
            + [pltpu.SemaphoreType.DMA] * (4 + 2 * nbuf)
            + [pltpu.VMEM_SHARED((NP, CH), jnp.float32)]
        ),
    )


_agg_l1 = _make_agg(1, False)
_agg_l2 = _make_agg(2, False)
_agg_l3 = _make_agg(1, True)


def _rsq(d):
    return lax.rsqrt(jnp.maximum(d, 1.0))


def _ln_relu_rows(t, g, b):
    mu = jnp.mean(t, axis=1, keepdims=True)
    var = jnp.mean((t - mu) ** 2, axis=1, keepdims=True)
    y = (t - mu) * lax.rsqrt(var + 1e-5) * g + b
    return jnp.maximum(y, 0.0)


def _prep_body(x_ref, ds_ref, o0, o1):
    ro = _rsq(ds_ref[...])
    h = x_ref[...] * ro
    o0[...] = h[:, :CH]
    o1[...] = h[:, CH:]


def _l1_body(a0, a1, w, b, g, be, ds, dd, o0, o1, o2, o3):
    agg = jnp.concatenate([a0[...], a1[...]], axis=1)
    rin = _rsq(dd[...])
    ro = _rsq(ds[...])
    t = jnp.dot(agg, w[...], preferred_element_type=jnp.float32) * rin + b[...]
    h = _ln_relu_rows(t, g[...], be[...]) * ro
    o0[...] = h[:, 0 * CH:1 * CH]
    o1[...] = h[:, 1 * CH:2 * CH]
    o2[...] = h[:, 2 * CH:3 * CH]
    o3[...] = h[:, 3 * CH:4 * CH]


def _l2_body(a0, a1, a2, a3, w2, b2, g2, be2, w3, ds, dd, op):
    agg = jnp.concatenate([a0[...], a1[...], a2[...], a3[...]], axis=1)
    rin = _rsq(dd[...])
    ro = _rsq(ds[...])
    t = jnp.dot(agg, w2[...], preferred_element_type=jnp.float32) * rin + b2[...]
    h = _ln_relu_rows(t, g2[...], be2[...]) * ro
    op[...] = jnp.dot(h, w3[...], preferred_element_type=jnp.float32)


def _final_body(p0, p1, b3, dd, o):
    rin = _rsq(dd[...])
    t = (p0[...] + p1[...]) * rin + b3[...]
    m = jnp.max(t, axis=1, keepdims=True)
    lse = jnp.log(jnp.sum(jnp.exp(t - m), axis=1, keepdims=True)) + m
    o[...] = t - lse


def _row_spec(d):
    return pl.BlockSpec((BM, d), lambda i: (i, 0))


def _full_spec(shape):
    return pl.BlockSpec(shape, lambda i: tuple(0 for _ in shape))


_COL = pl.BlockSpec((BM, 1), lambda i: (i, 0))
_GRID = (NP // BM,)


def _sds(shape):
    return jax.ShapeDtypeStruct(shape, jnp.float32)


_prep_call = pl.pallas_call(
    _prep_body,
    grid=_GRID,
    in_specs=[_row_spec(D_IN), _COL],
    out_specs=[_row_spec(CH), _row_spec(CH)],
    out_shape=[_sds((NP, CH))] * 2,
)

_l1_call = pl.pallas_call(
    _l1_body,
    grid=_GRID,
    in_specs=[_row_spec(CH), _row_spec(CH), _full_spec((D_IN, D_HID)),
              _full_spec((1, D_HID)), _full_spec((1, D_HID)),
              _full_spec((1, D_HID)), _COL, _COL],
    out_specs=[_row_spec(CH)] * 4,
    out_shape=[_sds((NP, CH))] * 4,
)

_l2_call = pl.pallas_call(
    _l2_body,
    grid=_GRID,
    in_specs=[_row_spec(CH)] * 4 + [_full_spec((D_HID, D_HID)),
              _full_spec((1, D_HID)), _full_spec((1, D_HID)),
              _full_spec((1, D_HID)), _full_spec((D_HID, N_CLS)),
              _COL, _COL],
    out_specs=_row_spec(N_CLS),
    out_shape=_sds((NP, N_CLS)),
)

_final_call = pl.pallas_call(
    _final_body,
    grid=_GRID,
    in_specs=[_row_spec(N_CLS), _row_spec(N_CLS),
              _full_spec((1, N_CLS)), _COL],
    out_specs=_row_spec(N_CLS),
    out_shape=_sds((NP, N_CLS)),
)


def kernel(x, edge_index, W1, b1, g1, be1, W2, b2, g2, be2, W3, b3):
    src = edge_index[0].astype(jnp.int32)
    dst = edge_index[1].astype(jnp.int32)
    pad = jnp.full((EP - E,), NP - 1, jnp.int32)
    srcp = jnp.concatenate([src, pad])
    dstp = jnp.concatenate([dst, pad])
    src16d = srcp.reshape(NT, EP // NT // DB, DB)
    dst16d = dstp.reshape(NT, EP // NT // DB, DB)
    src16 = srcp.reshape(NT, EP // NT // EB_FULL, EB_FULL)
    dst16 = dstp.reshape(NT, EP // NT // EB_FULL, EB_FULL)
    src32 = srcp.reshape(32, EP // 32 // EB_SPLIT, EB_SPLIT)
    dst32 = dstp.reshape(32, EP // 32 // EB_SPLIT, EB_SPLIT)
    xp = jnp.pad(x, ((0, NP - N_NODES), (0, 0)))

    deg_s, deg_d = _deg_call(src16d, dst16d)
    ds2 = deg_s.reshape(NP, 1)
    dd2 = deg_d.reshape(NP, 1)

    h0c0, h0c1 = _prep_call(xp, ds2)
    a10, a11 = _agg_l1(src16, dst16, h0c0, h0c1)
    hs = _l1_call(a10, a11, W1, b1.reshape(1, -1), g1.reshape(1, -1),
                  be1.reshape(1, -1), ds2, dd2)
    a2 = _agg_l2(src16, dst16, *hs)
    p = _l2_call(*a2, W2, b2.reshape(1, -1), g2.reshape(1, -1),
                 be2.reshape(1, -1), W3, ds2, dd2)
    p0, p1 = _agg_l3(src32, dst32, p)
    outp = _final_call(p0, p1, b3.reshape(1, -1), dd2)
    return outp[:N_NODES]
